# baseline (speedup 1.0000x reference)
import functools

import jax
import jax.numpy as jnp
from jax import lax
from jax.experimental import pallas as pl
from jax.experimental.pallas import tpu as pltpu, tpu_sc as plsc

N_NODE = 50000
N_USER = 20000
N_PRICE = 100
EMB = 64
CH = 128
NS = 16

_mesh = plsc.VectorSubcoreMesh(core_axis_name="c", subcore_axis_name="s")
_sc_params = pltpu.CompilerParams(
    needs_layout_passes=False, use_tc_tiling_on_sc=False)

E_ADJ = 802816
E_VU = 409600
E_UV = 409600
E_PV = 65536
E_VP = 65536

OFF_IU = 20000
OFF_PV = 40000
ACC_ROWS = N_NODE
USR_ROWS = 40128

ZR = 16


def _i16():
    return lax.iota(jnp.int32, 16)


def _zero_rows_2d(zrow_v, acc_s, base, nrows):
    full, rem = nrows // ZR, nrows % ZR

    def b(k, carry):
        pltpu.sync_copy(zrow_v, acc_s.at[pl.ds(base + k * ZR, ZR)])
        return carry
    lax.fori_loop(0, full, b, 0)
    if rem:
        pltpu.sync_copy(zrow_v.at[pl.ds(0, rem)],
                        acc_s.at[pl.ds(base + full * ZR, rem)])


def _zero_rows_1d(zvec_v, rs_s, base, n):
    full, rem = n // CH, n % CH

    def b(k, carry):
        pltpu.sync_copy(zvec_v, rs_s.at[pl.ds(base + k * CH, CH)])
        return carry
    lax.fori_loop(0, full, b, 0)
    if rem:
        pltpu.sync_copy(zvec_v.at[pl.ds(0, rem)],
                        rs_s.at[pl.ds(base + full * CH, rem)])


def _sc_layer_call(edges, emb2, usr2, pri2, idxl):
    (a_r, a_c, a_v, vu_r, vu_c, vu_v, uv_r, uv_c, uv_v,
     pv_r, pv_c, pv_v, vp_r, vp_c, vp_v) = edges

    def body(a_r_h, a_c_h, a_v_h, vu_r_h, vu_c_h, vu_v_h,
             uv_r_h, uv_c_h, uv_v_h, pv_r_h, pv_c_h, pv_v_h,
             vp_r_h, vp_c_h, vp_v_h, emb2_h, usr2_h, pri2_h, idxl_h,
             o_vp_h, o_adj_h, o_vu_h, o_usr_h,
             acc_s, rows_b, sidx_b, cols_bf, gidx_bf, vals_bf,
             pidx_v, pidxr_v, gath_a, gath_b, prod_a, prod_b, zrow_v, prif_v,
             sem_ga, sem_gb, sem_sa, sem_sb, sem2):
        c = lax.axis_index("c")
        s = lax.axis_index("s")

        z16 = jnp.zeros((16,), jnp.float32)
        for k in range(ZR):
            for h in range(2):
                zrow_v[k, pl.ds(16 * h, 16)] = z16

        def load_block(rows2_h, cols_h, vals_h, bb):
            pltpu.sync_copy(rows2_h.at[pl.ds(bb * 8, 8)], rows_b)
            pltpu.sync_copy(cols_h.at[pl.ds(bb * 1024, 1024)], cols_bf)
            pltpu.sync_copy(vals_h.at[pl.ds(bb * 1024, 1024)], vals_bf)

        def mk_gidx_block(c):
            def g(k, carry):
                for i in range(4):
                    cv = cols_bf[pl.ds(k * 64 + i * 16, 16)]
                    gidx_bf[pl.ds(k * 64 + i * 16, 16)] = cv * 2 + c
                return carry
            lax.fori_loop(0, 16, g, 0)

        def mul_to(gath_v, prod_v, k):
            def g(gi, carry):
                for eu in range(4):
                    e = gi * 4 + eu
                    vv = plsc.load_gather(
                        vals_bf, [jnp.full((16,), k * 128 + e, jnp.int32)])
                    for h in range(2):
                        prod_v[e, pl.ds(16 * h, 16)] = (
                            gath_v[e, pl.ds(16 * h, 16)] * vv)
                return carry
            lax.fori_loop(0, 32, g, 0)

        def mk_sidx_block(off):
            def g(k, carry):
                for i in range(8):
                    rv = rows_b[k, pl.ds(i * 16, 16)]
                    sidx_b[k, pl.ds(i * 16, 16)] = rv + off
                return carry
            lax.fori_loop(0, 8, g, 0)

        def spmm_stage(rows2_h, cols_h, vals_h, table_h, nblk, off, out_h,
                       zero=True):
            if zero:
                _zero_rows_2d(zrow_v, acc_s, s * (ACC_ROWS // NS),
                              ACC_ROWS // NS)
                plsc.subcore_barrier()

            def blk(b, carry):
                bb = s * nblk + b
                load_block(rows2_h, cols_h, vals_h, bb)
                mk_gidx_block(c)
                if off is not None:
                    mk_sidx_block(off)
                scat = sidx_b if off is not None else rows_b

                pend_g = [None, None]
                pend_s = [None, None]
                pend_g[0] = pltpu.async_copy(
                    table_h.at[gidx_bf.at[pl.ds(0, 128)]], gath_a, sem_ga)
                pend_g[1] = pltpu.async_copy(
                    table_h.at[gidx_bf.at[pl.ds(128, 128)]], gath_b, sem_gb)
                for k in range(8):
                    p = k % 2
                    gbuf, sg = (gath_a, sem_ga) if p == 0 else (gath_b, sem_gb)
                    pbuf, ss = (prod_a, sem_sa) if p == 0 else (prod_b, sem_sb)
                    pend_g[p].wait()
                    if pend_s[p] is not None:
                        pend_s[p].wait()
                    mul_to(gbuf, pbuf, k)
                    if k + 2 < 8:
                        pend_g[p] = pltpu.async_copy(
                            table_h.at[gidx_bf.at[pl.ds((k + 2) * 128, 128)]],
                            gbuf, sg)
                    pend_s[p] = pltpu.async_copy(
                        pbuf, acc_s.at[scat.at[k]], ss, add=True)
                pend_s[0].wait()
                pend_s[1].wait()
                return carry
            lax.fori_loop(0, nblk, blk, 0)
            plsc.subcore_barrier()
            if out_h is not None:
                n_rows = out_h.shape[1] // NS
                pltpu.sync_copy(acc_s.at[pl.ds(s * n_rows, n_rows)],
                                out_h.at[c, pl.ds(s * n_rows, n_rows)])
                plsc.subcore_barrier()

        spmm_stage(vp_r_h, vp_c_h, vp_v_h, pri2_h, E_VP // (NS * 1024),
                   None, o_vp_h)
        spmm_stage(a_r_h, a_c_h, a_v_h, emb2_h, E_ADJ // (NS * 1024),
                   None, o_adj_h)
        spmm_stage(vu_r_h, vu_c_h, vu_v_h, usr2_h, E_VU // (NS * 1024),
                   None, o_vu_h)

        _zero_rows_2d(zrow_v, acc_s, s * (ACC_ROWS // NS), ACC_ROWS // NS)

        def ld_pri(j, carry):
            pltpu.sync_copy(pri2_h.at[j], prif_v.at[pl.ds(j * 32, 32)])
            return carry
        lax.fori_loop(0, 2 * N_PRICE, ld_pri, 0)
        plsc.subcore_barrier()

        nblk_uv = E_UV // (NS * 1024)

        def blk_uv(b, carry):
            bb = s * nblk_uv + b
            load_block(uv_r_h, uv_c_h, uv_v_h, bb)
            mk_gidx_block(c)
            mk_sidx_block(OFF_IU)

            def chunk(k, carry2):
                cpa = pltpu.async_copy(
                    emb2_h.at[gidx_bf.at[pl.ds(k * 128, 128)]], gath_a, sem_ga)
                pltpu.async_copy(
                    idxl_h.at[cols_bf.at[pl.ds(k * 128, 128)]],
                    pidxr_v, sem2).wait()

                def mkp(g, carry3):
                    pv16 = pidxr_v[pl.ds(g * 16, 16)]
                    pidx_v[pl.ds(g * 16, 16)] = (pv16 * 2 + c) * 32
                    return carry3
                lax.fori_loop(0, 8, mkp, 0)

                def mulp(g, carry3):
                    for eu in range(16):
                        e = g * 16 + eu
                        e16 = jnp.full((16,), e, jnp.int32)
                        vv = plsc.load_gather(
                            vals_bf,
                            [jnp.full((16,), k * 128 + e, jnp.int32)])
                        pbase = plsc.load_gather(pidx_v, [e16])
                        for h in range(2):
                            gp = plsc.load_gather(
                                prif_v, [pbase + (_i16() + 16 * h)])
                            prod_b[e, pl.ds(16 * h, 16)] = gp * vv
                    return carry3
                lax.fori_loop(0, 8, mulp, 0)
                pltpu.sync_copy(prod_b, acc_s.at[rows_b.at[k]], add=True)

                cpa.wait()
                mul_to(gath_a, prod_a, k)
                pltpu.sync_copy(prod_a, acc_s.at[sidx_b.at[k]], add=True)
                return carry2
            lax.fori_loop(0, 8, chunk, 0)
            return carry
        lax.fori_loop(0, nblk_uv, blk_uv, 0)

        spmm_stage(pv_r_h, pv_c_h, pv_v_h, emb2_h, E_PV // (NS * 1024),
                   OFF_PV, None, zero=False)
        n_rows = USR_ROWS // NS
        pltpu.sync_copy(acc_s.at[pl.ds(s * n_rows, n_rows)],
                        o_usr_h.at[c, pl.ds(s * n_rows, n_rows)])

    return pl.kernel(
        body,
        out_type=(jax.ShapeDtypeStruct((2, N_NODE, 32), jnp.float32),
                  jax.ShapeDtypeStruct((2, N_NODE, 32), jnp.float32),
                  jax.ShapeDtypeStruct((2, N_NODE, 32), jnp.float32),
                  jax.ShapeDtypeStruct((2, USR_ROWS, 32), jnp.float32)),
        mesh=_mesh,
        compiler_params=_sc_params,
        scratch_types=[
            pltpu.VMEM_SHARED((ACC_ROWS, 32), jnp.float32),
            pltpu.VMEM((8, CH), jnp.int32),
            pltpu.VMEM((8, CH), jnp.int32),
            pltpu.VMEM((1024,), jnp.int32),
            pltpu.VMEM((1024,), jnp.int32),
            pltpu.VMEM((1024,), jnp.float32),
            pltpu.VMEM((CH,), jnp.int32),
            pltpu.VMEM((CH,), jnp.int32),
            pltpu.VMEM((CH, 32), jnp.float32),
            pltpu.VMEM((CH, 32), jnp.float32),
            pltpu.VMEM((CH, 32), jnp.float32),
            pltpu.VMEM((CH, 32), jnp.float32),
            pltpu.VMEM((ZR, 32), jnp.float32),
            pltpu.VMEM((2 * N_PRICE * 32,), jnp.float32),
            pltpu.SemaphoreType.DMA,
            pltpu.SemaphoreType.DMA,
            pltpu.SemaphoreType.DMA,
            pltpu.SemaphoreType.DMA,
            pltpu.SemaphoreType.DMA,
        ],
    )(a_r, a_c, a_v, vu_r, vu_c, vu_v, uv_r, uv_c, uv_v,
      pv_r, pv_c, pv_v, vp_r, vp_c, vp_v, emb2, usr2, pri2, idxl)


RS_V = 50048
RS_U = 20096
RS_P = 128


def _sc_rowsum_call(vu_r, vu_v, uv_r, uv_v, pv_r, pv_v):
    def body(vu_r_h, vu_v_h, uv_r_h, uv_v_h, pv_r_h, pv_v_h,
             rsv_o, rsu_o, rsp_o, rsv_s, rsu_s, rsp_s,
             rows_v, vals_v, zvec_v):
        c = lax.axis_index("c")
        s = lax.axis_index("s")
        z16 = jnp.zeros((16,), jnp.float32)
        for k in range(CH // 16):
            zvec_v[pl.ds(k * 16, 16)] = z16
        _zero_rows_1d(zvec_v, rsv_s, s * (RS_V // NS), RS_V // NS)
        _zero_rows_1d(zvec_v, rsu_s, s * (RS_U // NS), RS_U // NS)
        _zero_rows_1d(zvec_v, rsp_s, s * (RS_P // NS), RS_P // NS)
        plsc.subcore_barrier()

        def accum(rows2_h, vals_h, epad, rs_s):
            nch = epad // (2 * NS * CH)

            def chunk(j, carry):
                jj = c * (epad // (2 * CH)) + s * nch + j
                pltpu.sync_copy(vals_h.at[pl.ds(jj * CH, CH)], vals_v)
                pltpu.sync_copy(rows2_h.at[jj], rows_v)
                pltpu.sync_copy(vals_v, rs_s.at[rows_v], add=True)
                return carry
            lax.fori_loop(0, nch, chunk, 0)

        accum(vu_r_h, vu_v_h, E_VU, rsv_s)
        accum(uv_r_h, uv_v_h, E_UV, rsu_s)
        accum(pv_r_h, pv_v_h, E_PV, rsp_s)
        plsc.subcore_barrier()
        for rs_s, out_h, ext in ((rsv_s, rsv_o, RS_V), (rsu_s, rsu_o, RS_U),
                                 (rsp_s, rsp_o, RS_P)):
            q = ext // NS
            pltpu.sync_copy(rs_s.at[pl.ds(s * q, q)],
                            out_h.at[c, pl.ds(s * q, q)])

    return pl.kernel(
        body,
        out_type=(jax.ShapeDtypeStruct((2, RS_V), jnp.float32),
                  jax.ShapeDtypeStruct((2, RS_U), jnp.float32),
                  jax.ShapeDtypeStruct((2, RS_P), jnp.float32)),
        mesh=_mesh,
        compiler_params=_sc_params,
        scratch_types=[
            pltpu.VMEM_SHARED((RS_V,), jnp.float32),
            pltpu.VMEM_SHARED((RS_U,), jnp.float32),
            pltpu.VMEM_SHARED((RS_P,), jnp.float32),
            pltpu.VMEM((CH,), jnp.int32),
            pltpu.VMEM((CH,), jnp.float32),
            pltpu.VMEM((CH,), jnp.float32),
        ],
    )(vu_r, vu_v, uv_r, uv_v, pv_r, pv_v)


def _cat2(ref):
    return jnp.concatenate([ref[0], ref[1]], axis=-1)


def _dotT(x, w):
    return lax.dot_general(x, w, (((1,), (1,)), ((), ())),
                           preferred_element_type=jnp.float32)


def _tc_item_body(emb_ref, evp_ref, agg_ref, evu_ref, rsa_ref, rsb_ref, mat_ref,
                  wa_ref, ba_ref, wb_ref, bb_ref, out_ref):
    e1 = emb_ref[...]
    e2 = _cat2(evp_ref)
    x = jnp.concatenate([e1, e2], axis=-1)
    gate = jax.nn.sigmoid(_dotT(x, wa_ref[...]) + ba_ref[...]
                          + _dotT(e1, wb_ref[...]) + bb_ref[...])
    rs = rsa_ref[...] + rsb_ref[...] + 1e-8
    evu = _cat2(evu_ref)
    agg = _cat2(agg_ref)
    out_ref[...] = e1 + gate * e2 + agg + (evu / rs) * mat_ref[...]


def _tc_item(emb, evp, agg, evu, rsa, rsb, mat_l, Wa, ba, Wb, bb):
    bs = 1000
    grid = N_NODE // bs
    return pl.pallas_call(
        _tc_item_body,
        grid=(grid,),
        in_specs=[
            pl.BlockSpec((bs, EMB), lambda i: (i, 0)),
            pl.BlockSpec((2, bs, 32), lambda i: (0, i, 0)),
            pl.BlockSpec((2, bs, 32), lambda i: (0, i, 0)),
            pl.BlockSpec((2, bs, 32), lambda i: (0, i, 0)),
            pl.BlockSpec((bs, 1), lambda i: (i, 0)),
            pl.BlockSpec((bs, 1), lambda i: (i, 0)),
            pl.BlockSpec((bs, 1), lambda i: (i, 0)),
            pl.BlockSpec((EMB, 2 * EMB), lambda i: (0, 0)),
            pl.BlockSpec((1, EMB), lambda i: (0, 0)),
            pl.BlockSpec((EMB, EMB), lambda i: (0, 0)),
            pl.BlockSpec((1, EMB), lambda i: (0, 0)),
        ],
        out_specs=pl.BlockSpec((bs, EMB), lambda i: (i, 0)),
        out_shape=jax.ShapeDtypeStruct((N_NODE, EMB), jnp.float32),
    )(emb, evp, agg, evu, rsa, rsb, mat_l, Wa, ba, Wb, bb)


def _tc_price_body(pri_ref, epv_ref, rsa_ref, rsb_ref, mat_ref,
                   wa_ref, ba_ref, wb_ref, bb_ref, out_ref):
    e1 = pri_ref[...]
    rs = rsa_ref[...] + rsb_ref[...] + 1e-8
    e2 = (_cat2(epv_ref) / rs) * mat_ref[...]
    x = jnp.concatenate([e1, e2], axis=-1)
    gate = jax.nn.sigmoid(_dotT(x, wa_ref[...]) + ba_ref[...]
                          + _dotT(e1, wb_ref[...]) + bb_ref[...])
    out_ref[...] = e1 + gate * e2


def _tc_price(pri_pad, epv, rsa, rsb, mat_pad, Wa, ba, Wb, bb):
    return pl.pallas_call(
        _tc_price_body,
        out_shape=jax.ShapeDtypeStruct((RS_P, EMB), jnp.float32),
    )(pri_pad, epv, rsa, rsb, mat_pad, Wa, ba, Wb, bb)


def _tc_user_body(usr_ref, epu_ref, eiu_ref, rsa_ref, rsb_ref, mat_ref,
                  w_ref, b_ref, out_ref):
    u = usr_ref[...]
    rs = rsa_ref[...] + rsb_ref[...] + 1e-8
    eiu = (_cat2(eiu_ref) / rs) * mat_ref[...]
    epu = _cat2(epu_ref)
    x = jnp.concatenate([u, eiu, epu], axis=-1)
    gate = jax.nn.sigmoid(_dotT(x, w_ref[...]) + b_ref[...])
    out_ref[...] = gate * u + (1.0 - gate) * eiu


def _tc_user(usr, epu, eiu, rsa, rsb, mat_l, W, b):
    bs = 1000
    grid = N_USER // bs
    return pl.pallas_call(
        _tc_user_body,
        grid=(grid,),
        in_specs=[
            pl.BlockSpec((bs, EMB), lambda i: (i, 0)),
            pl.BlockSpec((2, bs, 32), lambda i: (0, i, 0)),
            pl.BlockSpec((2, bs, 32), lambda i: (0, i, 0)),
            pl.BlockSpec((bs, 1), lambda i: (i, 0)),
            pl.BlockSpec((bs, 1), lambda i: (i, 0)),
            pl.BlockSpec((bs, 1), lambda i: (i, 0)),
            pl.BlockSpec((EMB, 3 * EMB), lambda i: (0, 0)),
            pl.BlockSpec((1, EMB), lambda i: (0, 0)),
        ],
        out_specs=pl.BlockSpec((bs, EMB), lambda i: (i, 0)),
        out_shape=jax.ShapeDtypeStruct((N_USER, EMB), jnp.float32),
    )(usr, epu, eiu, rsa, rsb, mat_l, W, b)


def _pad_edges(r, c, v, epad):
    n = r.shape[0]
    return (jnp.pad(r, (0, epad - n)), jnp.pad(c, (0, epad - n)),
            jnp.pad(v, (0, epad - n)))


def kernel(adj_rows, adj_cols, adj_vals, adj_pv_rows, adj_pv_cols, adj_pv_vals, adj_vp_rows, adj_vp_cols, adj_vp_vals, adj_uv_rows, adj_uv_cols, adj_uv_vals, adj_vu_rows, adj_vu_cols, adj_vu_vals, adj_pc_rows, adj_pc_cols, adj_pc_vals, adj_cp_rows, adj_cp_cols, adj_cp_vals, adj_cv_rows, adj_cv_cols, adj_cv_vals, adj_vc_rows, adj_vc_cols, adj_vc_vals, embedding, pri_emb, cate_emb, user_emb, mat_vu, mat_pv, mat_uv, Wa_i, ba_i, Wb_i, bb_i, Wa_p, ba_p, Wb_p, bb_p, W_user, b_user, user_lambda):
    a_r, a_c, a_v = _pad_edges(adj_rows, adj_cols, adj_vals, E_ADJ)
    vu_r, vu_c, vu_v = _pad_edges(adj_vu_rows, adj_vu_cols, adj_vu_vals, E_VU)
    uv_r, uv_c, uv_v = _pad_edges(adj_uv_rows, adj_uv_cols, adj_uv_vals, E_UV)
    pv_r, pv_c, pv_v = _pad_edges(adj_pv_rows, adj_pv_cols, adj_pv_vals, E_PV)
    vp_r, vp_c, vp_v = _pad_edges(adj_vp_rows, adj_vp_cols, adj_vp_vals, E_VP)
    a_r, vu_r, uv_r, pv_r, vp_r = (x.reshape(-1, CH)
                                   for x in (a_r, vu_r, uv_r, pv_r, vp_r))
    edges = (a_r, a_c, a_v, vu_r, vu_c, vu_v, uv_r, uv_c, uv_v,
             pv_r, pv_c, pv_v, vp_r, vp_c, vp_v)

    rsv2, rsu2, rsp2 = _sc_rowsum_call(vu_r, vu_v, uv_r, uv_v, pv_r, pv_v)
    rsv2 = rsv2[:, :N_NODE]
    rsu2 = rsu2[:, :N_USER]

    emb2 = embedding.reshape(2 * N_NODE, 32)
    usr2 = user_emb.reshape(2 * N_USER, 32)
    pri_pad = jnp.pad(pri_emb, ((0, RS_P - N_PRICE), (0, 0)))
    pri2 = pri_pad.reshape(2 * RS_P, 32)
    emb_cur = embedding
    usr_cur = user_emb

    mat_vu_l = user_lambda * mat_vu
    mat_pv_pad = jnp.pad(mat_pv, ((0, RS_P - N_PRICE), (0, 0)))
    mat_uv_l = mat_uv
    ba2, bb2 = ba_i.reshape(1, EMB), bb_i.reshape(1, EMB)
    bap2, bbp2 = ba_p.reshape(1, EMB), bb_p.reshape(1, EMB)
    bu2 = b_user.reshape(1, EMB)

    price_key = jax.random.key(42)
    for i in range(2):
        idx = jax.random.randint(jax.random.fold_in(price_key, i),
                                 (N_NODE,), 0, N_PRICE, dtype=jnp.int32)
        idxl = jnp.pad(idx, (0, 50048 - N_NODE))
        o_vp, o_adj, o_vu, o_usr = _sc_layer_call(edges, emb2, usr2, pri2, idxl)
        epu = o_usr[:, :OFF_IU]
        eiu = o_usr[:, OFF_IU:OFF_PV]
        epv = o_usr[:, OFF_PV:OFF_PV + RS_P]

        item = _tc_item(emb_cur, o_vp, o_adj, o_vu, rsv2[0][:, None],
                        rsv2[1][:, None], mat_vu_l, Wa_i, ba2, Wb_i, bb2)
        price_pad = _tc_price(pri_pad, epv, rsp2[0][:, None], rsp2[1][:, None],
                              mat_pv_pad, Wa_p, bap2, Wb_p, bbp2)
        user = _tc_user(usr_cur, epu, eiu, rsu2[0][:, None], rsu2[1][:, None],
                        mat_uv_l, W_user, bu2)

        emb_cur, pri_pad, usr_cur = item, price_pad, user
        emb2 = emb_cur.reshape(2 * N_NODE, 32)
        usr2 = usr_cur.reshape(2 * N_USER, 32)
        pri2 = pri_pad.reshape(2 * RS_P, 32)

    return (emb_cur, pri_pad[:N_PRICE], usr_cur)


# speedup vs baseline: 1.3130x; 1.3130x over previous
import functools

import jax
import jax.numpy as jnp
from jax import lax
from jax.experimental import pallas as pl
from jax.experimental.pallas import tpu as pltpu, tpu_sc as plsc

N_NODE = 50000
N_USER = 20000
N_PRICE = 100
EMB = 64
CH = 128
NS = 16

_mesh = plsc.VectorSubcoreMesh(core_axis_name="c", subcore_axis_name="s")
_sc_params = pltpu.CompilerParams(
    needs_layout_passes=False, use_tc_tiling_on_sc=False)

E_ADJ = 802816
E_VU = 409600
E_UV = 409600
E_PV = 65536
E_VP = 65536

OFF_IU = 20000
OFF_PV = 40000
ACC_ROWS = N_NODE
USR_ROWS = 40128

ZR = 16


def _i16():
    return lax.iota(jnp.int32, 16)


def _zero_rows_2d(zrow_v, acc_s, base, nrows):
    full, rem = nrows // ZR, nrows % ZR

    def b(k, carry):
        pltpu.sync_copy(zrow_v, acc_s.at[pl.ds(base + k * ZR, ZR)])
        return carry
    lax.fori_loop(0, full, b, 0)
    if rem:
        pltpu.sync_copy(zrow_v.at[pl.ds(0, rem)],
                        acc_s.at[pl.ds(base + full * ZR, rem)])


def _zero_rows_1d(zvec_v, rs_s, base, n):
    full, rem = n // CH, n % CH

    def b(k, carry):
        pltpu.sync_copy(zvec_v, rs_s.at[pl.ds(base + k * CH, CH)])
        return carry
    lax.fori_loop(0, full, b, 0)
    if rem:
        pltpu.sync_copy(zvec_v.at[pl.ds(0, rem)],
                        rs_s.at[pl.ds(base + full * CH, rem)])


def _sc_layer_call(edges, emb2, usr2, pri2, idxl):
    (a_r, a_c, a_v, vu_r, vu_c, vu_v, uv_r, uv_c, uv_v,
     pv_r, pv_c, pv_v, vp_r, vp_c, vp_v) = edges

    def body(a_r_h, a_c_h, a_v_h, vu_r_h, vu_c_h, vu_v_h,
             uv_r_h, uv_c_h, uv_v_h, pv_r_h, pv_c_h, pv_v_h,
             vp_r_h, vp_c_h, vp_v_h, emb2_h, usr2_h, pri2_h, idxl_h,
             o_vp_h, o_adj_h, o_vu_h, o_usr_h,
             acc_s, rows_b, sidx_b, cols_bf, gidx_bf, vals_bf,
             pidx_v, pidxr_v, gath_a, gath_b, prod_a, prod_b, zrow_v, prif_v,
             sem_ga, sem_gb, sem_sa, sem_sb, sem2):
        c = lax.axis_index("c")
        s = lax.axis_index("s")

        z16 = jnp.zeros((16,), jnp.float32)
        for k in range(ZR):
            for h in range(2):
                zrow_v[k, pl.ds(16 * h, 16)] = z16

        def load_block(rows2_h, cols_h, vals_h, bb):
            pltpu.sync_copy(rows2_h.at[pl.ds(bb * 8, 8)], rows_b)
            pltpu.sync_copy(cols_h.at[pl.ds(bb * 1024, 1024)], cols_bf)
            pltpu.sync_copy(vals_h.at[pl.ds(bb * 1024, 1024)], vals_bf)

        def mk_gidx_block(c):
            @plsc.parallel_loop(0, 64, unroll=4)
            def _(g):
                cv = cols_bf[pl.ds(g * 16, 16)]
                gidx_bf[pl.ds(g * 16, 16)] = cv * 2 + c

        def mul_to(gath_v, prod_v, k):
            @plsc.parallel_loop(0, CH, unroll=8)
            def _(e):
                vv = plsc.load_gather(
                    vals_bf, [jnp.full((16,), k * CH + e, jnp.int32)])
                for h in range(2):
                    prod_v[e, pl.ds(16 * h, 16)] = (
                        gath_v[e, pl.ds(16 * h, 16)] * vv)

        def mk_sidx_block(off):
            @plsc.parallel_loop(0, 8, unroll=2)
            def _(k):
                for i in range(8):
                    rv = rows_b[k, pl.ds(i * 16, 16)]
                    sidx_b[k, pl.ds(i * 16, 16)] = rv + off

        def spmm_stage(rows2_h, cols_h, vals_h, table_h, nblk, off, out_h,
                       zero=True):
            if zero:
                _zero_rows_2d(zrow_v, acc_s, s * (ACC_ROWS // NS),
                              ACC_ROWS // NS)
                plsc.subcore_barrier()

            def blk(b, carry):
                bb = s * nblk + b
                load_block(rows2_h, cols_h, vals_h, bb)
                mk_gidx_block(c)
                if off is not None:
                    mk_sidx_block(off)
                scat = sidx_b if off is not None else rows_b

                def pair(kk, carry2):
                    k0 = kk * 2
                    k1 = k0 + 1
                    cpa = pltpu.async_copy(
                        table_h.at[gidx_bf.at[pl.ds(k0 * 128, 128)]],
                        gath_a, sem_ga)
                    cpb = pltpu.async_copy(
                        table_h.at[gidx_bf.at[pl.ds(k1 * 128, 128)]],
                        gath_b, sem_gb)
                    cpa.wait()
                    mul_to(gath_a, prod_a, k0)
                    cps = pltpu.async_copy(
                        prod_a, acc_s.at[scat.at[k0]], sem_sa, add=True)
                    cpb.wait()
                    mul_to(gath_b, prod_b, k1)
                    pltpu.sync_copy(prod_b, acc_s.at[scat.at[k1]], add=True)
                    cps.wait()
                    return carry2
                lax.fori_loop(0, 4, pair, 0)
                return carry
            lax.fori_loop(0, nblk, blk, 0)
            plsc.subcore_barrier()
            if out_h is not None:
                n_rows = out_h.shape[1] // NS
                pltpu.sync_copy(acc_s.at[pl.ds(s * n_rows, n_rows)],
                                out_h.at[c, pl.ds(s * n_rows, n_rows)])
                plsc.subcore_barrier()

        spmm_stage(vp_r_h, vp_c_h, vp_v_h, pri2_h, E_VP // (NS * 1024),
                   None, o_vp_h)
        spmm_stage(a_r_h, a_c_h, a_v_h, emb2_h, E_ADJ // (NS * 1024),
                   None, o_adj_h)
        spmm_stage(vu_r_h, vu_c_h, vu_v_h, usr2_h, E_VU // (NS * 1024),
                   None, o_vu_h)

        _zero_rows_2d(zrow_v, acc_s, s * (ACC_ROWS // NS), ACC_ROWS // NS)

        def ld_pri(j, carry):
            pltpu.sync_copy(pri2_h.at[j], prif_v.at[pl.ds(j * 32, 32)])
            return carry
        lax.fori_loop(0, 2 * N_PRICE, ld_pri, 0)
        plsc.subcore_barrier()

        nblk_uv = E_UV // (NS * 1024)

        def blk_uv(b, carry):
            bb = s * nblk_uv + b
            load_block(uv_r_h, uv_c_h, uv_v_h, bb)
            mk_gidx_block(c)
            mk_sidx_block(OFF_IU)

            def chunk(k, carry2):
                cpa = pltpu.async_copy(
                    emb2_h.at[gidx_bf.at[pl.ds(k * 128, 128)]], gath_a, sem_ga)
                pltpu.async_copy(
                    idxl_h.at[cols_bf.at[pl.ds(k * 128, 128)]],
                    pidxr_v, sem2).wait()

                @plsc.parallel_loop(0, 8, unroll=2)
                def _mkp(g):
                    pv16 = pidxr_v[pl.ds(g * 16, 16)]
                    pidx_v[pl.ds(g * 16, 16)] = (pv16 * 2 + c) * 32

                @plsc.parallel_loop(0, CH, unroll=8)
                def _(e):
                    e16 = jnp.full((16,), e, jnp.int32)
                    vv = plsc.load_gather(
                        vals_bf, [jnp.full((16,), k * CH + e, jnp.int32)])
                    pbase = plsc.load_gather(pidx_v, [e16])
                    for h in range(2):
                        gp = plsc.load_gather(
                            prif_v, [pbase + (_i16() + 16 * h)])
                        prod_b[e, pl.ds(16 * h, 16)] = gp * vv
                pltpu.sync_copy(prod_b, acc_s.at[rows_b.at[k]], add=True)

                cpa.wait()
                mul_to(gath_a, prod_a, k)
                pltpu.sync_copy(prod_a, acc_s.at[sidx_b.at[k]], add=True)
                return carry2
            lax.fori_loop(0, 8, chunk, 0)
            return carry
        lax.fori_loop(0, nblk_uv, blk_uv, 0)

        spmm_stage(pv_r_h, pv_c_h, pv_v_h, emb2_h, E_PV // (NS * 1024),
                   OFF_PV, None, zero=False)
        n_rows = USR_ROWS // NS
        pltpu.sync_copy(acc_s.at[pl.ds(s * n_rows, n_rows)],
                        o_usr_h.at[c, pl.ds(s * n_rows, n_rows)])

    return pl.kernel(
        body,
        out_type=(jax.ShapeDtypeStruct((2, N_NODE, 32), jnp.float32),
                  jax.ShapeDtypeStruct((2, N_NODE, 32), jnp.float32),
                  jax.ShapeDtypeStruct((2, N_NODE, 32), jnp.float32),
                  jax.ShapeDtypeStruct((2, USR_ROWS, 32), jnp.float32)),
        mesh=_mesh,
        compiler_params=_sc_params,
        scratch_types=[
            pltpu.VMEM_SHARED((ACC_ROWS, 32), jnp.float32),
            pltpu.VMEM((8, CH), jnp.int32),
            pltpu.VMEM((8, CH), jnp.int32),
            pltpu.VMEM((1024,), jnp.int32),
            pltpu.VMEM((1024,), jnp.int32),
            pltpu.VMEM((1024,), jnp.float32),
            pltpu.VMEM((CH,), jnp.int32),
            pltpu.VMEM((CH,), jnp.int32),
            pltpu.VMEM((CH, 32), jnp.float32),
            pltpu.VMEM((CH, 32), jnp.float32),
            pltpu.VMEM((CH, 32), jnp.float32),
            pltpu.VMEM((CH, 32), jnp.float32),
            pltpu.VMEM((ZR, 32), jnp.float32),
            pltpu.VMEM((2 * N_PRICE * 32,), jnp.float32),
            pltpu.SemaphoreType.DMA,
            pltpu.SemaphoreType.DMA,
            pltpu.SemaphoreType.DMA,
            pltpu.SemaphoreType.DMA,
            pltpu.SemaphoreType.DMA,
        ],
    )(a_r, a_c, a_v, vu_r, vu_c, vu_v, uv_r, uv_c, uv_v,
      pv_r, pv_c, pv_v, vp_r, vp_c, vp_v, emb2, usr2, pri2, idxl)


RS_V = 50048
RS_U = 20096
RS_P = 128


def _sc_rowsum_call(vu_r, vu_v, uv_r, uv_v, pv_r, pv_v):
    def body(vu_r_h, vu_v_h, uv_r_h, uv_v_h, pv_r_h, pv_v_h,
             rsv_o, rsu_o, rsp_o, rsv_s, rsu_s, rsp_s,
             rows_v, vals_v, zvec_v):
        c = lax.axis_index("c")
        s = lax.axis_index("s")
        z16 = jnp.zeros((16,), jnp.float32)
        for k in range(CH // 16):
            zvec_v[pl.ds(k * 16, 16)] = z16
        _zero_rows_1d(zvec_v, rsv_s, s * (RS_V // NS), RS_V // NS)
        _zero_rows_1d(zvec_v, rsu_s, s * (RS_U // NS), RS_U // NS)
        _zero_rows_1d(zvec_v, rsp_s, s * (RS_P // NS), RS_P // NS)
        plsc.subcore_barrier()

        def accum(rows2_h, vals_h, epad, rs_s):
            nch = epad // (2 * NS * CH)

            def chunk(j, carry):
                jj = c * (epad // (2 * CH)) + s * nch + j
                pltpu.sync_copy(vals_h.at[pl.ds(jj * CH, CH)], vals_v)
                pltpu.sync_copy(rows2_h.at[jj], rows_v)
                pltpu.sync_copy(vals_v, rs_s.at[rows_v], add=True)
                return carry
            lax.fori_loop(0, nch, chunk, 0)

        accum(vu_r_h, vu_v_h, E_VU, rsv_s)
        accum(uv_r_h, uv_v_h, E_UV, rsu_s)
        accum(pv_r_h, pv_v_h, E_PV, rsp_s)
        plsc.subcore_barrier()
        for rs_s, out_h, ext in ((rsv_s, rsv_o, RS_V), (rsu_s, rsu_o, RS_U),
                                 (rsp_s, rsp_o, RS_P)):
            q = ext // NS
            pltpu.sync_copy(rs_s.at[pl.ds(s * q, q)],
                            out_h.at[c, pl.ds(s * q, q)])

    return pl.kernel(
        body,
        out_type=(jax.ShapeDtypeStruct((2, RS_V), jnp.float32),
                  jax.ShapeDtypeStruct((2, RS_U), jnp.float32),
                  jax.ShapeDtypeStruct((2, RS_P), jnp.float32)),
        mesh=_mesh,
        compiler_params=_sc_params,
        scratch_types=[
            pltpu.VMEM_SHARED((RS_V,), jnp.float32),
            pltpu.VMEM_SHARED((RS_U,), jnp.float32),
            pltpu.VMEM_SHARED((RS_P,), jnp.float32),
            pltpu.VMEM((CH,), jnp.int32),
            pltpu.VMEM((CH,), jnp.float32),
            pltpu.VMEM((CH,), jnp.float32),
        ],
    )(vu_r, vu_v, uv_r, uv_v, pv_r, pv_v)


def _cat2(ref):
    return jnp.concatenate([ref[0], ref[1]], axis=-1)


def _dotT(x, w):
    return lax.dot_general(x, w, (((1,), (1,)), ((), ())),
                           preferred_element_type=jnp.float32)


def _tc_item_body(emb_ref, evp_ref, agg_ref, evu_ref, rsa_ref, rsb_ref, mat_ref,
                  wa_ref, ba_ref, wb_ref, bb_ref, out_ref):
    e1 = emb_ref[...]
    e2 = _cat2(evp_ref)
    x = jnp.concatenate([e1, e2], axis=-1)
    gate = jax.nn.sigmoid(_dotT(x, wa_ref[...]) + ba_ref[...]
                          + _dotT(e1, wb_ref[...]) + bb_ref[...])
    rs = rsa_ref[...] + rsb_ref[...] + 1e-8
    evu = _cat2(evu_ref)
    agg = _cat2(agg_ref)
    out_ref[...] = e1 + gate * e2 + agg + (evu / rs) * mat_ref[...]


def _tc_item(emb, evp, agg, evu, rsa, rsb, mat_l, Wa, ba, Wb, bb):
    bs = 1000
    grid = N_NODE // bs
    return pl.pallas_call(
        _tc_item_body,
        grid=(grid,),
        in_specs=[
            pl.BlockSpec((bs, EMB), lambda i: (i, 0)),
            pl.BlockSpec((2, bs, 32), lambda i: (0, i, 0)),
            pl.BlockSpec((2, bs, 32), lambda i: (0, i, 0)),
            pl.BlockSpec((2, bs, 32), lambda i: (0, i, 0)),
            pl.BlockSpec((bs, 1), lambda i: (i, 0)),
            pl.BlockSpec((bs, 1), lambda i: (i, 0)),
            pl.BlockSpec((bs, 1), lambda i: (i, 0)),
            pl.BlockSpec((EMB, 2 * EMB), lambda i: (0, 0)),
            pl.BlockSpec((1, EMB), lambda i: (0, 0)),
            pl.BlockSpec((EMB, EMB), lambda i: (0, 0)),
            pl.BlockSpec((1, EMB), lambda i: (0, 0)),
        ],
        out_specs=pl.BlockSpec((bs, EMB), lambda i: (i, 0)),
        out_shape=jax.ShapeDtypeStruct((N_NODE, EMB), jnp.float32),
    )(emb, evp, agg, evu, rsa, rsb, mat_l, Wa, ba, Wb, bb)


def _tc_price_body(pri_ref, epv_ref, rsa_ref, rsb_ref, mat_ref,
                   wa_ref, ba_ref, wb_ref, bb_ref, out_ref):
    e1 = pri_ref[...]
    rs = rsa_ref[...] + rsb_ref[...] + 1e-8
    e2 = (_cat2(epv_ref) / rs) * mat_ref[...]
    x = jnp.concatenate([e1, e2], axis=-1)
    gate = jax.nn.sigmoid(_dotT(x, wa_ref[...]) + ba_ref[...]
                          + _dotT(e1, wb_ref[...]) + bb_ref[...])
    out_ref[...] = e1 + gate * e2


def _tc_price(pri_pad, epv, rsa, rsb, mat_pad, Wa, ba, Wb, bb):
    return pl.pallas_call(
        _tc_price_body,
        out_shape=jax.ShapeDtypeStruct((RS_P, EMB), jnp.float32),
    )(pri_pad, epv, rsa, rsb, mat_pad, Wa, ba, Wb, bb)


def _tc_user_body(usr_ref, epu_ref, eiu_ref, rsa_ref, rsb_ref, mat_ref,
                  w_ref, b_ref, out_ref):
    u = usr_ref[...]
    rs = rsa_ref[...] + rsb_ref[...] + 1e-8
    eiu = (_cat2(eiu_ref) / rs) * mat_ref[...]
    epu = _cat2(epu_ref)
    x = jnp.concatenate([u, eiu, epu], axis=-1)
    gate = jax.nn.sigmoid(_dotT(x, w_ref[...]) + b_ref[...])
    out_ref[...] = gate * u + (1.0 - gate) * eiu


def _tc_user(usr, epu, eiu, rsa, rsb, mat_l, W, b):
    bs = 1000
    grid = N_USER // bs
    return pl.pallas_call(
        _tc_user_body,
        grid=(grid,),
        in_specs=[
            pl.BlockSpec((bs, EMB), lambda i: (i, 0)),
            pl.BlockSpec((2, bs, 32), lambda i: (0, i, 0)),
            pl.BlockSpec((2, bs, 32), lambda i: (0, i, 0)),
            pl.BlockSpec((bs, 1), lambda i: (i, 0)),
            pl.BlockSpec((bs, 1), lambda i: (i, 0)),
            pl.BlockSpec((bs, 1), lambda i: (i, 0)),
            pl.BlockSpec((EMB, 3 * EMB), lambda i: (0, 0)),
            pl.BlockSpec((1, EMB), lambda i: (0, 0)),
        ],
        out_specs=pl.BlockSpec((bs, EMB), lambda i: (i, 0)),
        out_shape=jax.ShapeDtypeStruct((N_USER, EMB), jnp.float32),
    )(usr, epu, eiu, rsa, rsb, mat_l, W, b)


def _pad_edges(r, c, v, epad):
    n = r.shape[0]
    return (jnp.pad(r, (0, epad - n)), jnp.pad(c, (0, epad - n)),
            jnp.pad(v, (0, epad - n)))


def kernel(adj_rows, adj_cols, adj_vals, adj_pv_rows, adj_pv_cols, adj_pv_vals, adj_vp_rows, adj_vp_cols, adj_vp_vals, adj_uv_rows, adj_uv_cols, adj_uv_vals, adj_vu_rows, adj_vu_cols, adj_vu_vals, adj_pc_rows, adj_pc_cols, adj_pc_vals, adj_cp_rows, adj_cp_cols, adj_cp_vals, adj_cv_rows, adj_cv_cols, adj_cv_vals, adj_vc_rows, adj_vc_cols, adj_vc_vals, embedding, pri_emb, cate_emb, user_emb, mat_vu, mat_pv, mat_uv, Wa_i, ba_i, Wb_i, bb_i, Wa_p, ba_p, Wb_p, bb_p, W_user, b_user, user_lambda):
    a_r, a_c, a_v = _pad_edges(adj_rows, adj_cols, adj_vals, E_ADJ)
    vu_r, vu_c, vu_v = _pad_edges(adj_vu_rows, adj_vu_cols, adj_vu_vals, E_VU)
    uv_r, uv_c, uv_v = _pad_edges(adj_uv_rows, adj_uv_cols, adj_uv_vals, E_UV)
    pv_r, pv_c, pv_v = _pad_edges(adj_pv_rows, adj_pv_cols, adj_pv_vals, E_PV)
    vp_r, vp_c, vp_v = _pad_edges(adj_vp_rows, adj_vp_cols, adj_vp_vals, E_VP)
    a_r, vu_r, uv_r, pv_r, vp_r = (x.reshape(-1, CH)
                                   for x in (a_r, vu_r, uv_r, pv_r, vp_r))
    edges = (a_r, a_c, a_v, vu_r, vu_c, vu_v, uv_r, uv_c, uv_v,
             pv_r, pv_c, pv_v, vp_r, vp_c, vp_v)

    rsv2, rsu2, rsp2 = _sc_rowsum_call(vu_r, vu_v, uv_r, uv_v, pv_r, pv_v)
    rsv2 = rsv2[:, :N_NODE]
    rsu2 = rsu2[:, :N_USER]

    emb2 = embedding.reshape(2 * N_NODE, 32)
    usr2 = user_emb.reshape(2 * N_USER, 32)
    pri_pad = jnp.pad(pri_emb, ((0, RS_P - N_PRICE), (0, 0)))
    pri2 = pri_pad.reshape(2 * RS_P, 32)
    emb_cur = embedding
    usr_cur = user_emb

    mat_vu_l = user_lambda * mat_vu
    mat_pv_pad = jnp.pad(mat_pv, ((0, RS_P - N_PRICE), (0, 0)))
    mat_uv_l = mat_uv
    ba2, bb2 = ba_i.reshape(1, EMB), bb_i.reshape(1, EMB)
    bap2, bbp2 = ba_p.reshape(1, EMB), bb_p.reshape(1, EMB)
    bu2 = b_user.reshape(1, EMB)

    price_key = jax.random.key(42)
    for i in range(2):
        idx = jax.random.randint(jax.random.fold_in(price_key, i),
                                 (N_NODE,), 0, N_PRICE, dtype=jnp.int32)
        idxl = jnp.pad(idx, (0, 50048 - N_NODE))
        o_vp, o_adj, o_vu, o_usr = _sc_layer_call(edges, emb2, usr2, pri2, idxl)
        epu = o_usr[:, :OFF_IU]
        eiu = o_usr[:, OFF_IU:OFF_PV]
        epv = o_usr[:, OFF_PV:OFF_PV + RS_P]

        item = _tc_item(emb_cur, o_vp, o_adj, o_vu, rsv2[0][:, None],
                        rsv2[1][:, None], mat_vu_l, Wa_i, ba2, Wb_i, bb2)
        price_pad = _tc_price(pri_pad, epv, rsp2[0][:, None], rsp2[1][:, None],
                              mat_pv_pad, Wa_p, bap2, Wb_p, bbp2)
        user = _tc_user(usr_cur, epu, eiu, rsu2[0][:, None], rsu2[1][:, None],
                        mat_uv_l, W_user, bu2)

        emb_cur, pri_pad, usr_cur = item, price_pad, user
        emb2 = emb_cur.reshape(2 * N_NODE, 32)
        usr2 = usr_cur.reshape(2 * N_USER, 32)
        pri2 = pri_pad.reshape(2 * RS_P, 32)

    return (emb_cur, pri_pad[:N_PRICE], usr_cur)


# speedup vs baseline: 1.3626x; 1.0378x over previous
import functools

import jax
import jax.numpy as jnp
from jax import lax
from jax.experimental import pallas as pl
from jax.experimental.pallas import tpu as pltpu, tpu_sc as plsc

N_NODE = 50000
N_USER = 20000
N_PRICE = 100
EMB = 64
CH = 128
NS = 16

_mesh = plsc.VectorSubcoreMesh(core_axis_name="c", subcore_axis_name="s")
_sc_params = pltpu.CompilerParams(
    needs_layout_passes=False, use_tc_tiling_on_sc=False)

E_ADJ = 802816
E_VU = 409600
E_UV = 409600
E_PV = 65536
E_VP = 65536

OFF_IU = 20000
OFF_PV = 40000
ACC_ROWS = N_NODE
USR_ROWS = 40128

ZR = 16


def _i16():
    return lax.iota(jnp.int32, 16)


def _zero_rows_2d(zrow_v, acc_s, base, nrows):
    full, rem = nrows // ZR, nrows % ZR

    def b(k, carry):
        pltpu.sync_copy(zrow_v, acc_s.at[pl.ds(base + k * ZR, ZR)])
        return carry
    lax.fori_loop(0, full, b, 0)
    if rem:
        pltpu.sync_copy(zrow_v.at[pl.ds(0, rem)],
                        acc_s.at[pl.ds(base + full * ZR, rem)])


def _zero_rows_1d(zvec_v, rs_s, base, n):
    full, rem = n // CH, n % CH

    def b(k, carry):
        pltpu.sync_copy(zvec_v, rs_s.at[pl.ds(base + k * CH, CH)])
        return carry
    lax.fori_loop(0, full, b, 0)
    if rem:
        pltpu.sync_copy(zvec_v.at[pl.ds(0, rem)],
                        rs_s.at[pl.ds(base + full * CH, rem)])


def _sc_layer_call(edges, emb2, usr2, pri2, idxl):
    (a_r, a_c, a_v, vu_r, vu_c, vu_v, uv_r, uv_c, uv_v,
     pv_r, pv_c, pv_v, vp_r, vp_c, vp_v) = edges

    def body(a_r_h, a_c_h, a_v_h, vu_r_h, vu_c_h, vu_v_h,
             uv_r_h, uv_c_h, uv_v_h, pv_r_h, pv_c_h, pv_v_h,
             vp_r_h, vp_c_h, vp_v_h, emb2_h, usr2_h, pri2_h, idxl_h,
             o_vp_h, o_adj_h, o_vu_h, o_usr_h,
             acc_s, rows_b, sidx_b, cols_bf, gidx_bf, vals_bf,
             pidx_v, pidxr_v, gath_a, gath_b, prod_a, prod_b, zrow_v, prif_v,
             sem_ga, sem_gb, sem_sa, sem_sb, sem2):
        c = lax.axis_index("c")
        s = lax.axis_index("s")

        z16 = jnp.zeros((16,), jnp.float32)
        for k in range(ZR):
            for h in range(2):
                zrow_v[k, pl.ds(16 * h, 16)] = z16

        def load_block(rows2_h, cols_h, vals_h, bb):
            c1 = pltpu.async_copy(rows2_h.at[pl.ds(bb * 8, 8)], rows_b, sem_ga)
            c2 = pltpu.async_copy(cols_h.at[pl.ds(bb * 1024, 1024)], cols_bf,
                                  sem_gb)
            c3 = pltpu.async_copy(vals_h.at[pl.ds(bb * 1024, 1024)], vals_bf,
                                  sem_sa)
            c1.wait()
            c2.wait()
            c3.wait()

        def mk_gidx_block(c):
            @plsc.parallel_loop(0, 64, unroll=8)
            def _(g):
                cv = cols_bf[pl.ds(g * 16, 16)]
                gidx_bf[pl.ds(g * 16, 16)] = cv * 2 + c

        def mul_to(gath_v, prod_v, k):
            @plsc.parallel_loop(0, CH, unroll=16)
            def _(e):
                vv = plsc.load_gather(
                    vals_bf, [jnp.full((16,), k * CH + e, jnp.int32)])
                for h in range(2):
                    prod_v[e, pl.ds(16 * h, 16)] = (
                        gath_v[e, pl.ds(16 * h, 16)] * vv)

        def mk_sidx_block(off):
            @plsc.parallel_loop(0, 8, unroll=4)
            def _(k):
                for i in range(8):
                    rv = rows_b[k, pl.ds(i * 16, 16)]
                    sidx_b[k, pl.ds(i * 16, 16)] = rv + off

        def spmm_stage(rows2_h, cols_h, vals_h, table_h, nblk, off, out_h,
                       zero=True):
            if zero:
                _zero_rows_2d(zrow_v, acc_s, s * (ACC_ROWS // NS),
                              ACC_ROWS // NS)
                plsc.subcore_barrier()

            def blk(b, carry):
                bb = s * nblk + b
                load_block(rows2_h, cols_h, vals_h, bb)
                mk_gidx_block(c)
                if off is not None:
                    mk_sidx_block(off)
                scat = sidx_b if off is not None else rows_b

                def pair(kk, carry2):
                    k0 = kk * 2
                    k1 = k0 + 1
                    cpa = pltpu.async_copy(
                        table_h.at[gidx_bf.at[pl.ds(k0 * 128, 128)]],
                        gath_a, sem_ga)
                    cpb = pltpu.async_copy(
                        table_h.at[gidx_bf.at[pl.ds(k1 * 128, 128)]],
                        gath_b, sem_gb)
                    cpa.wait()
                    mul_to(gath_a, prod_a, k0)
                    cps = pltpu.async_copy(
                        prod_a, acc_s.at[scat.at[k0]], sem_sa, add=True)
                    cpb.wait()
                    mul_to(gath_b, prod_b, k1)
                    pltpu.sync_copy(prod_b, acc_s.at[scat.at[k1]], add=True)
                    cps.wait()
                    return carry2
                lax.fori_loop(0, 4, pair, 0)
                return carry
            lax.fori_loop(0, nblk, blk, 0)
            plsc.subcore_barrier()
            if out_h is not None:
                n_rows = out_h.shape[1] // NS
                pltpu.sync_copy(acc_s.at[pl.ds(s * n_rows, n_rows)],
                                out_h.at[c, pl.ds(s * n_rows, n_rows)])
                plsc.subcore_barrier()

        spmm_stage(vp_r_h, vp_c_h, vp_v_h, pri2_h, E_VP // (NS * 1024),
                   None, o_vp_h)
        spmm_stage(a_r_h, a_c_h, a_v_h, emb2_h, E_ADJ // (NS * 1024),
                   None, o_adj_h)
        spmm_stage(vu_r_h, vu_c_h, vu_v_h, usr2_h, E_VU // (NS * 1024),
                   None, o_vu_h)

        _zero_rows_2d(zrow_v, acc_s, s * (ACC_ROWS // NS), ACC_ROWS // NS)

        def ld_pri(j, carry):
            pltpu.sync_copy(pri2_h.at[j], prif_v.at[pl.ds(j * 32, 32)])
            return carry
        lax.fori_loop(0, 2 * N_PRICE, ld_pri, 0)
        plsc.subcore_barrier()

        nblk_uv = E_UV // (NS * 1024)

        def blk_uv(b, carry):
            bb = s * nblk_uv + b
            load_block(uv_r_h, uv_c_h, uv_v_h, bb)
            mk_gidx_block(c)
            mk_sidx_block(OFF_IU)

            def chunk(k, carry2):
                cpa = pltpu.async_copy(
                    emb2_h.at[gidx_bf.at[pl.ds(k * 128, 128)]], gath_a, sem_ga)
                pltpu.async_copy(
                    idxl_h.at[cols_bf.at[pl.ds(k * 128, 128)]],
                    pidxr_v, sem2).wait()

                @plsc.parallel_loop(0, 8, unroll=2)
                def _mkp(g):
                    pv16 = pidxr_v[pl.ds(g * 16, 16)]
                    pidx_v[pl.ds(g * 16, 16)] = (pv16 * 2 + c) * 32

                @plsc.parallel_loop(0, CH, unroll=16)
                def _(e):
                    e16 = jnp.full((16,), e, jnp.int32)
                    vv = plsc.load_gather(
                        vals_bf, [jnp.full((16,), k * CH + e, jnp.int32)])
                    pbase = plsc.load_gather(pidx_v, [e16])
                    for h in range(2):
                        gp = plsc.load_gather(
                            prif_v, [pbase + (_i16() + 16 * h)])
                        prod_b[e, pl.ds(16 * h, 16)] = gp * vv
                pltpu.sync_copy(prod_b, acc_s.at[rows_b.at[k]], add=True)

                cpa.wait()
                mul_to(gath_a, prod_a, k)
                pltpu.sync_copy(prod_a, acc_s.at[sidx_b.at[k]], add=True)
                return carry2
            lax.fori_loop(0, 8, chunk, 0)
            return carry
        lax.fori_loop(0, nblk_uv, blk_uv, 0)

        spmm_stage(pv_r_h, pv_c_h, pv_v_h, emb2_h, E_PV // (NS * 1024),
                   OFF_PV, None, zero=False)
        n_rows = USR_ROWS // NS
        pltpu.sync_copy(acc_s.at[pl.ds(s * n_rows, n_rows)],
                        o_usr_h.at[c, pl.ds(s * n_rows, n_rows)])

    return pl.kernel(
        body,
        out_type=(jax.ShapeDtypeStruct((2, N_NODE, 32), jnp.float32),
                  jax.ShapeDtypeStruct((2, N_NODE, 32), jnp.float32),
                  jax.ShapeDtypeStruct((2, N_NODE, 32), jnp.float32),
                  jax.ShapeDtypeStruct((2, USR_ROWS, 32), jnp.float32)),
        mesh=_mesh,
        compiler_params=_sc_params,
        scratch_types=[
            pltpu.VMEM_SHARED((ACC_ROWS, 32), jnp.float32),
            pltpu.VMEM((8, CH), jnp.int32),
            pltpu.VMEM((8, CH), jnp.int32),
            pltpu.VMEM((1024,), jnp.int32),
            pltpu.VMEM((1024,), jnp.int32),
            pltpu.VMEM((1024,), jnp.float32),
            pltpu.VMEM((CH,), jnp.int32),
            pltpu.VMEM((CH,), jnp.int32),
            pltpu.VMEM((CH, 32), jnp.float32),
            pltpu.VMEM((CH, 32), jnp.float32),
            pltpu.VMEM((CH, 32), jnp.float32),
            pltpu.VMEM((CH, 32), jnp.float32),
            pltpu.VMEM((ZR, 32), jnp.float32),
            pltpu.VMEM((2 * N_PRICE * 32,), jnp.float32),
            pltpu.SemaphoreType.DMA,
            pltpu.SemaphoreType.DMA,
            pltpu.SemaphoreType.DMA,
            pltpu.SemaphoreType.DMA,
            pltpu.SemaphoreType.DMA,
        ],
    )(a_r, a_c, a_v, vu_r, vu_c, vu_v, uv_r, uv_c, uv_v,
      pv_r, pv_c, pv_v, vp_r, vp_c, vp_v, emb2, usr2, pri2, idxl)


RS_V = 50048
RS_U = 20096
RS_P = 128


def _sc_rowsum_call(vu_r, vu_v, uv_r, uv_v, pv_r, pv_v):
    def body(vu_r_h, vu_v_h, uv_r_h, uv_v_h, pv_r_h, pv_v_h,
             rsv_o, rsu_o, rsp_o, rsv_s, rsu_s, rsp_s,
             rows_v, vals_v, zvec_v):
        c = lax.axis_index("c")
        s = lax.axis_index("s")
        z16 = jnp.zeros((16,), jnp.float32)
        for k in range(CH // 16):
            zvec_v[pl.ds(k * 16, 16)] = z16
        _zero_rows_1d(zvec_v, rsv_s, s * (RS_V // NS), RS_V // NS)
        _zero_rows_1d(zvec_v, rsu_s, s * (RS_U // NS), RS_U // NS)
        _zero_rows_1d(zvec_v, rsp_s, s * (RS_P // NS), RS_P // NS)
        plsc.subcore_barrier()

        def accum(rows2_h, vals_h, epad, rs_s):
            nch = epad // (2 * NS * CH)

            def chunk(j, carry):
                jj = c * (epad // (2 * CH)) + s * nch + j
                pltpu.sync_copy(vals_h.at[pl.ds(jj * CH, CH)], vals_v)
                pltpu.sync_copy(rows2_h.at[jj], rows_v)
                pltpu.sync_copy(vals_v, rs_s.at[rows_v], add=True)
                return carry
            lax.fori_loop(0, nch, chunk, 0)

        accum(vu_r_h, vu_v_h, E_VU, rsv_s)
        accum(uv_r_h, uv_v_h, E_UV, rsu_s)
        accum(pv_r_h, pv_v_h, E_PV, rsp_s)
        plsc.subcore_barrier()
        for rs_s, out_h, ext in ((rsv_s, rsv_o, RS_V), (rsu_s, rsu_o, RS_U),
                                 (rsp_s, rsp_o, RS_P)):
            q = ext // NS
            pltpu.sync_copy(rs_s.at[pl.ds(s * q, q)],
                            out_h.at[c, pl.ds(s * q, q)])

    return pl.kernel(
        body,
        out_type=(jax.ShapeDtypeStruct((2, RS_V), jnp.float32),
                  jax.ShapeDtypeStruct((2, RS_U), jnp.float32),
                  jax.ShapeDtypeStruct((2, RS_P), jnp.float32)),
        mesh=_mesh,
        compiler_params=_sc_params,
        scratch_types=[
            pltpu.VMEM_SHARED((RS_V,), jnp.float32),
            pltpu.VMEM_SHARED((RS_U,), jnp.float32),
            pltpu.VMEM_SHARED((RS_P,), jnp.float32),
            pltpu.VMEM((CH,), jnp.int32),
            pltpu.VMEM((CH,), jnp.float32),
            pltpu.VMEM((CH,), jnp.float32),
        ],
    )(vu_r, vu_v, uv_r, uv_v, pv_r, pv_v)


def _cat2(ref):
    return jnp.concatenate([ref[0], ref[1]], axis=-1)


def _dotT(x, w):
    return lax.dot_general(x, w, (((1,), (1,)), ((), ())),
                           preferred_element_type=jnp.float32)


def _tc_item_body(emb_ref, evp_ref, agg_ref, evu_ref, rsa_ref, rsb_ref, mat_ref,
                  wa_ref, ba_ref, wb_ref, bb_ref, out_ref):
    e1 = emb_ref[...]
    e2 = _cat2(evp_ref)
    x = jnp.concatenate([e1, e2], axis=-1)
    gate = jax.nn.sigmoid(_dotT(x, wa_ref[...]) + ba_ref[...]
                          + _dotT(e1, wb_ref[...]) + bb_ref[...])
    rs = rsa_ref[...] + rsb_ref[...] + 1e-8
    evu = _cat2(evu_ref)
    agg = _cat2(agg_ref)
    out_ref[...] = e1 + gate * e2 + agg + (evu / rs) * mat_ref[...]


def _tc_item(emb, evp, agg, evu, rsa, rsb, mat_l, Wa, ba, Wb, bb):
    bs = 1000
    grid = N_NODE // bs
    return pl.pallas_call(
        _tc_item_body,
        grid=(grid,),
        in_specs=[
            pl.BlockSpec((bs, EMB), lambda i: (i, 0)),
            pl.BlockSpec((2, bs, 32), lambda i: (0, i, 0)),
            pl.BlockSpec((2, bs, 32), lambda i: (0, i, 0)),
            pl.BlockSpec((2, bs, 32), lambda i: (0, i, 0)),
            pl.BlockSpec((bs, 1), lambda i: (i, 0)),
            pl.BlockSpec((bs, 1), lambda i: (i, 0)),
            pl.BlockSpec((bs, 1), lambda i: (i, 0)),
            pl.BlockSpec((EMB, 2 * EMB), lambda i: (0, 0)),
            pl.BlockSpec((1, EMB), lambda i: (0, 0)),
            pl.BlockSpec((EMB, EMB), lambda i: (0, 0)),
            pl.BlockSpec((1, EMB), lambda i: (0, 0)),
        ],
        out_specs=pl.BlockSpec((bs, EMB), lambda i: (i, 0)),
        out_shape=jax.ShapeDtypeStruct((N_NODE, EMB), jnp.float32),
    )(emb, evp, agg, evu, rsa, rsb, mat_l, Wa, ba, Wb, bb)


def _tc_price_body(pri_ref, epv_ref, rsa_ref, rsb_ref, mat_ref,
                   wa_ref, ba_ref, wb_ref, bb_ref, out_ref):
    e1 = pri_ref[...]
    rs = rsa_ref[...] + rsb_ref[...] + 1e-8
    e2 = (_cat2(epv_ref) / rs) * mat_ref[...]
    x = jnp.concatenate([e1, e2], axis=-1)
    gate = jax.nn.sigmoid(_dotT(x, wa_ref[...]) + ba_ref[...]
                          + _dotT(e1, wb_ref[...]) + bb_ref[...])
    out_ref[...] = e1 + gate * e2


def _tc_price(pri_pad, epv, rsa, rsb, mat_pad, Wa, ba, Wb, bb):
    return pl.pallas_call(
        _tc_price_body,
        out_shape=jax.ShapeDtypeStruct((RS_P, EMB), jnp.float32),
    )(pri_pad, epv, rsa, rsb, mat_pad, Wa, ba, Wb, bb)


def _tc_user_body(usr_ref, epu_ref, eiu_ref, rsa_ref, rsb_ref, mat_ref,
                  w_ref, b_ref, out_ref):
    u = usr_ref[...]
    rs = rsa_ref[...] + rsb_ref[...] + 1e-8
    eiu = (_cat2(eiu_ref) / rs) * mat_ref[...]
    epu = _cat2(epu_ref)
    x = jnp.concatenate([u, eiu, epu], axis=-1)
    gate = jax.nn.sigmoid(_dotT(x, w_ref[...]) + b_ref[...])
    out_ref[...] = gate * u + (1.0 - gate) * eiu


def _tc_user(usr, epu, eiu, rsa, rsb, mat_l, W, b):
    bs = 1000
    grid = N_USER // bs
    return pl.pallas_call(
        _tc_user_body,
        grid=(grid,),
        in_specs=[
            pl.BlockSpec((bs, EMB), lambda i: (i, 0)),
            pl.BlockSpec((2, bs, 32), lambda i: (0, i, 0)),
            pl.BlockSpec((2, bs, 32), lambda i: (0, i, 0)),
            pl.BlockSpec((bs, 1), lambda i: (i, 0)),
            pl.BlockSpec((bs, 1), lambda i: (i, 0)),
            pl.BlockSpec((bs, 1), lambda i: (i, 0)),
            pl.BlockSpec((EMB, 3 * EMB), lambda i: (0, 0)),
            pl.BlockSpec((1, EMB), lambda i: (0, 0)),
        ],
        out_specs=pl.BlockSpec((bs, EMB), lambda i: (i, 0)),
        out_shape=jax.ShapeDtypeStruct((N_USER, EMB), jnp.float32),
    )(usr, epu, eiu, rsa, rsb, mat_l, W, b)


def _pad_edges(r, c, v, epad):
    n = r.shape[0]
    return (jnp.pad(r, (0, epad - n)), jnp.pad(c, (0, epad - n)),
            jnp.pad(v, (0, epad - n)))


def kernel(adj_rows, adj_cols, adj_vals, adj_pv_rows, adj_pv_cols, adj_pv_vals, adj_vp_rows, adj_vp_cols, adj_vp_vals, adj_uv_rows, adj_uv_cols, adj_uv_vals, adj_vu_rows, adj_vu_cols, adj_vu_vals, adj_pc_rows, adj_pc_cols, adj_pc_vals, adj_cp_rows, adj_cp_cols, adj_cp_vals, adj_cv_rows, adj_cv_cols, adj_cv_vals, adj_vc_rows, adj_vc_cols, adj_vc_vals, embedding, pri_emb, cate_emb, user_emb, mat_vu, mat_pv, mat_uv, Wa_i, ba_i, Wb_i, bb_i, Wa_p, ba_p, Wb_p, bb_p, W_user, b_user, user_lambda):
    a_r, a_c, a_v = _pad_edges(adj_rows, adj_cols, adj_vals, E_ADJ)
    vu_r, vu_c, vu_v = _pad_edges(adj_vu_rows, adj_vu_cols, adj_vu_vals, E_VU)
    uv_r, uv_c, uv_v = _pad_edges(adj_uv_rows, adj_uv_cols, adj_uv_vals, E_UV)
    pv_r, pv_c, pv_v = _pad_edges(adj_pv_rows, adj_pv_cols, adj_pv_vals, E_PV)
    vp_r, vp_c, vp_v = _pad_edges(adj_vp_rows, adj_vp_cols, adj_vp_vals, E_VP)
    a_r, vu_r, uv_r, pv_r, vp_r = (x.reshape(-1, CH)
                                   for x in (a_r, vu_r, uv_r, pv_r, vp_r))
    edges = (a_r, a_c, a_v, vu_r, vu_c, vu_v, uv_r, uv_c, uv_v,
             pv_r, pv_c, pv_v, vp_r, vp_c, vp_v)

    rsv2, rsu2, rsp2 = _sc_rowsum_call(vu_r, vu_v, uv_r, uv_v, pv_r, pv_v)
    rsv2 = rsv2[:, :N_NODE]
    rsu2 = rsu2[:, :N_USER]

    emb2 = embedding.reshape(2 * N_NODE, 32)
    usr2 = user_emb.reshape(2 * N_USER, 32)
    pri_pad = jnp.pad(pri_emb, ((0, RS_P - N_PRICE), (0, 0)))
    pri2 = pri_pad.reshape(2 * RS_P, 32)
    emb_cur = embedding
    usr_cur = user_emb

    mat_vu_l = user_lambda * mat_vu
    mat_pv_pad = jnp.pad(mat_pv, ((0, RS_P - N_PRICE), (0, 0)))
    mat_uv_l = mat_uv
    ba2, bb2 = ba_i.reshape(1, EMB), bb_i.reshape(1, EMB)
    bap2, bbp2 = ba_p.reshape(1, EMB), bb_p.reshape(1, EMB)
    bu2 = b_user.reshape(1, EMB)

    price_key = jax.random.key(42)
    for i in range(2):
        idx = jax.random.randint(jax.random.fold_in(price_key, i),
                                 (N_NODE,), 0, N_PRICE, dtype=jnp.int32)
        idxl = jnp.pad(idx, (0, 50048 - N_NODE))
        o_vp, o_adj, o_vu, o_usr = _sc_layer_call(edges, emb2, usr2, pri2, idxl)
        epu = o_usr[:, :OFF_IU]
        eiu = o_usr[:, OFF_IU:OFF_PV]
        epv = o_usr[:, OFF_PV:OFF_PV + RS_P]

        item = _tc_item(emb_cur, o_vp, o_adj, o_vu, rsv2[0][:, None],
                        rsv2[1][:, None], mat_vu_l, Wa_i, ba2, Wb_i, bb2)
        price_pad = _tc_price(pri_pad, epv, rsp2[0][:, None], rsp2[1][:, None],
                              mat_pv_pad, Wa_p, bap2, Wb_p, bbp2)
        user = _tc_user(usr_cur, epu, eiu, rsu2[0][:, None], rsu2[1][:, None],
                        mat_uv_l, W_user, bu2)

        emb_cur, pri_pad, usr_cur = item, price_pad, user
        emb2 = emb_cur.reshape(2 * N_NODE, 32)
        usr2 = usr_cur.reshape(2 * N_USER, 32)
        pri2 = pri_pad.reshape(2 * RS_P, 32)

    return (emb_cur, pri_pad[:N_PRICE], usr_cur)
